# baseline (speedup 1.0000x reference)
import functools

import jax
import jax.numpy as jnp
from jax.experimental import pallas as pl
from jax.experimental.pallas import tpu as pltpu

_M = 0.2
_S = 30.0
_LOG2E = 1.4426950408889634
_K1 = _S * _LOG2E

_W = 512
_G = 14


def _tc_body(ng, c, *refs):
    x_refs = refs[:_G]
    t_ref, o_ref, acc, tacc = refs[_G:]
    nc = ng * _G
    j = pl.program_id(0)

    @pl.when(j == 0)
    def _():
        acc[...] = jnp.zeros_like(acc)
        tacc[...] = jnp.zeros_like(tacc)

    t = t_ref[...]
    iota = jax.lax.broadcasted_iota(jnp.int32, (1, _W), 1)

    @pl.when(j < ng - 1)
    def _():
        s = jnp.zeros_like(acc)
        xt = jnp.zeros_like(tacc)
        for g in range(_G):
            x = x_refs[g][...]
            e = jnp.exp2(x * _K1)
            s += jnp.sum(e, axis=1, keepdims=True)
            cols = (g * ng + j) * _W + iota
            xt += jnp.sum(jnp.where(cols == t, x, 0.0), axis=1,
                          keepdims=True)
        acc[...] += s
        tacc[...] += xt

    @pl.when(j == ng - 1)
    def _():
        s = acc[...]
        xt = tacc[...]
        for g in range(_G):
            x = x_refs[g][...]
            cols = (g * ng + j) * _W + iota
            e = jnp.exp2(x * _K1)
            if g == _G - 1:
                e = jnp.where(cols < c, e, 0.0)
            s += jnp.sum(e, axis=1, keepdims=True)
            xt += jnp.sum(jnp.where(cols == t, x, 0.0), axis=1,
                          keepdims=True)
        e_old = jnp.exp2(xt * _K1)
        e_new = jnp.exp2((xt - _M) * _K1)
        s_mod = s - e_old + e_new
        loss = jnp.log(s_mod) - _S * (xt - _M)
        o_ref[...] = jnp.mean(loss, keepdims=True)


def kernel(inputs, targets):
    b, c = inputs.shape
    nc = pl.cdiv(c, _W)
    ng = nc // _G
    t2 = targets.reshape(b, 1)
    in_specs = [
        pl.BlockSpec((b, _W), functools.partial(
            lambda g, j: (0, g * ng + j), g))
        for g in range(_G)
    ]
    in_specs.append(pl.BlockSpec((b, 1), lambda j: (0, 0)))
    out = pl.pallas_call(
        functools.partial(_tc_body, ng, c),
        grid=(ng,),
        in_specs=in_specs,
        out_specs=pl.BlockSpec((1, 1), lambda j: (0, 0)),
        out_shape=jax.ShapeDtypeStruct((1, 1), jnp.float32),
        scratch_shapes=[
            pltpu.VMEM((b, 1), jnp.float32),
            pltpu.VMEM((b, 1), jnp.float32),
        ],
        compiler_params=pltpu.CompilerParams(
            vmem_limit_bytes=100 * 1024 * 1024),
    )(*([inputs] * _G), t2)
    return out[0, 0]


# speedup vs baseline: 1.0009x; 1.0009x over previous
import functools

import jax
import jax.numpy as jnp
from jax.experimental import pallas as pl
from jax.experimental.pallas import tpu as pltpu

_M = 0.2
_S = 30.0
_LOG2E = 1.4426950408889634
_K1 = _S * _LOG2E

_W = 1024
_G = 7


def _tc_body(ng, c, *refs):
    x_refs = refs[:_G]
    t_ref, o_ref, acc, tacc = refs[_G:]
    nc = ng * _G
    j = pl.program_id(0)

    @pl.when(j == 0)
    def _():
        acc[...] = jnp.zeros_like(acc)
        tacc[...] = jnp.zeros_like(tacc)

    t = t_ref[...]
    iota = jax.lax.broadcasted_iota(jnp.int32, (1, _W), 1)

    @pl.when(j < ng - 1)
    def _():
        s = jnp.zeros_like(acc)
        xt = jnp.zeros_like(tacc)
        for g in range(_G):
            x = x_refs[g][...]
            e = jnp.exp2(x * _K1)
            s += jnp.sum(e, axis=1, keepdims=True)
            cols = (g * ng + j) * _W + iota
            xt += jnp.sum(jnp.where(cols == t, x, 0.0), axis=1,
                          keepdims=True)
        acc[...] += s
        tacc[...] += xt

    @pl.when(j == ng - 1)
    def _():
        s = acc[...]
        xt = tacc[...]
        for g in range(_G):
            x = x_refs[g][...]
            cols = (g * ng + j) * _W + iota
            e = jnp.exp2(x * _K1)
            if g == _G - 1:
                e = jnp.where(cols < c, e, 0.0)
            s += jnp.sum(e, axis=1, keepdims=True)
            xt += jnp.sum(jnp.where(cols == t, x, 0.0), axis=1,
                          keepdims=True)
        e_old = jnp.exp2(xt * _K1)
        e_new = jnp.exp2((xt - _M) * _K1)
        s_mod = s - e_old + e_new
        loss = jnp.log(s_mod) - _S * (xt - _M)
        o_ref[...] = jnp.mean(loss, keepdims=True)


def kernel(inputs, targets):
    b, c = inputs.shape
    nc = pl.cdiv(c, _W)
    ng = nc // _G
    t2 = targets.reshape(b, 1)
    in_specs = [
        pl.BlockSpec((b, _W), functools.partial(
            lambda g, j: (0, g * ng + j), g))
        for g in range(_G)
    ]
    in_specs.append(pl.BlockSpec((b, 1), lambda j: (0, 0)))
    out = pl.pallas_call(
        functools.partial(_tc_body, ng, c),
        grid=(ng,),
        in_specs=in_specs,
        out_specs=pl.BlockSpec((1, 1), lambda j: (0, 0)),
        out_shape=jax.ShapeDtypeStruct((1, 1), jnp.float32),
        scratch_shapes=[
            pltpu.VMEM((b, 1), jnp.float32),
            pltpu.VMEM((b, 1), jnp.float32),
        ],
        compiler_params=pltpu.CompilerParams(
            vmem_limit_bytes=100 * 1024 * 1024),
    )(*([inputs] * _G), t2)
    return out[0, 0]
